# baseline (speedup 1.0000x reference)
import functools

import jax
import jax.numpy as jnp
import numpy as np
from jax import lax
from jax.experimental import pallas as pl
from jax.experimental.pallas import tpu as pltpu
from jax.experimental.pallas import tpu_sc as plsc

N_USERS = 4000
N_ITEMS = 6000
N_NODES = N_USERS + N_ITEMS
N_EDGES = 320000
D_FEAT = 128
D_PK = D_FEAT // 2

NC = 2
NS = 16
NW = NC * NS
CHUNK = 128
N_CHUNKS = N_EDGES // CHUNK
CPT = 80
N_CHUNKS_PAD = CPT * NW
ROW_GROUPS = N_NODES // 8
GROUPS_PER_TILE = ROW_GROUPS // NS
GROUP_EXTRA = ROW_GROUPS - GROUPS_PER_TILE * NS

_PERM = np.empty(D_FEAT, dtype=np.int32)
for _t in range(4):
    for _i in range(16):
        _PERM[32 * _t + 2 * _i] = 32 * _t + _i
        _PERM[32 * _t + 2 * _i + 1] = 32 * _t + 16 + _i

_sc_mesh = plsc.VectorSubcoreMesh(core_axis_name="c", subcore_axis_name="s")


def _row_range(s):
    row0 = 8 * (s * GROUPS_PER_TILE + jnp.minimum(s, GROUP_EXTRA))
    return pl.multiple_of(row0, 8)


@functools.partial(
    pl.kernel,
    out_type=jax.ShapeDtypeStruct((NC, 1, N_NODES), jnp.float32),
    mesh=_sc_mesh,
    scratch_types=[
        pltpu.VMEM_SHARED((N_NODES,), jnp.float32),
        pltpu.VMEM((CPT, CHUNK), jnp.int32),
        pltpu.VMEM((CHUNK,), jnp.float32),
        pltpu.SemaphoreType.DMA,
    ],
)
def _deg_kernel(dst2_hbm, zeros1_hbm, out_hbm, deg_sh, dst2d, ones_v, sem):
    c = lax.axis_index("c")
    s = lax.axis_index("s")
    wid = s * NC + c

    pltpu.sync_copy(dst2_hbm.at[pl.ds(CPT * wid, CPT)], dst2d)

    @pl.when(s == 0)
    def _():
        pltpu.sync_copy(zeros1_hbm, deg_sh)

    for j in range(CHUNK // 16):
        ones_v[pl.ds(j * 16, 16)] = jnp.ones((16,), jnp.float32)
    plsc.subcore_barrier()

    for j in range(CPT):
        @pl.when(CPT * wid + j < N_CHUNKS)
        def _():
            pltpu.async_copy(ones_v, deg_sh.at[dst2d.at[j]], sem, add=True)
    for j in range(CPT):
        @pl.when(CPT * wid + j < N_CHUNKS)
        def _():
            pltpu.make_async_copy(zeros1_hbm.at[pl.ds(0, CHUNK)], ones_v, sem).wait()

    plsc.subcore_barrier()

    @pl.when(s == 0)
    def _():
        pltpu.sync_copy(deg_sh, out_hbm.at[c, 0])


@functools.partial(
    pl.kernel,
    out_type=jax.ShapeDtypeStruct((NC, N_NODES, D_FEAT), jnp.float32),
    mesh=_sc_mesh,
    scratch_types=[
        pltpu.VMEM_SHARED((N_NODES, D_FEAT), jnp.float32),
        pltpu.VMEM((4, CHUNK), jnp.int32),
        pltpu.VMEM((4, CHUNK), jnp.int32),
        pltpu.VMEM((CHUNK, D_PK), jnp.int32),
        pltpu.VMEM((CHUNK, D_PK), jnp.int32),
        pltpu.VMEM((CHUNK, D_FEAT), jnp.float32),
        pltpu.VMEM((CHUNK, D_FEAT), jnp.float32),
        pltpu.SemaphoreType.DMA,
        pltpu.SemaphoreType.DMA,
        pltpu.SemaphoreType.DMA,
        pltpu.SemaphoreType.DMA,
        pltpu.SemaphoreType.DMA,
        pltpu.SemaphoreType.DMA,
        pltpu.SemaphoreType.DMA,
        pltpu.SemaphoreType.DMA,
    ],
    compiler_params=pltpu.CompilerParams(use_tc_tiling_on_sc=False),
)
def _hop_kernel(ypk_hbm, srcf_hbm, dstf_hbm, out_hbm,
                acc_sh, sidr, didr, pk0, pk1, rf0, rf1,
                is0, is1, is2, is3,
                gsem0, gsem1, ssem0, ssem1):
    isems = (is0, is1, is2, is3)
    c = lax.axis_index("c")
    s = lax.axis_index("s")
    wid = s * NC + c
    base0 = CPT * wid * CHUNK

    pks = (pk0, pk1)
    rfs = (rf0, rf1)
    gsems = (gsem0, gsem1)
    ssems = (ssem0, ssem1)

    def idx_load(j, slot):
        b = pl.multiple_of(base0 + j * CHUNK, CHUNK)
        pltpu.async_copy(srcf_hbm.at[pl.ds(b, CHUNK)], sidr.at[slot],
                         isems[slot])
        pltpu.async_copy(dstf_hbm.at[pl.ds(b, CHUNK)], didr.at[slot],
                         isems[slot])

    def idx_wait(slot):
        pltpu.make_async_copy(srcf_hbm.at[pl.ds(0, CHUNK)], sidr.at[slot],
                              isems[slot]).wait()
        pltpu.make_async_copy(dstf_hbm.at[pl.ds(0, CHUNK)], didr.at[slot],
                              isems[slot]).wait()

    def zero_row(i, carry):
        for k in range(D_FEAT // 16):
            rf0[i, pl.ds(k * 16, 16)] = jnp.zeros((16,), jnp.float32)
        return carry

    lax.fori_loop(0, CHUNK, zero_row, 0)

    idx_load(0, 0)
    idx_load(1, 1)
    idx_load(2, 2)

    row0 = _row_range(s)

    def zero_acc(t, carry):
        pltpu.sync_copy(rf0, acc_sh.at[pl.ds(row0 + t * CHUNK, CHUNK)])
        return carry

    lax.fori_loop(0, 4, zero_acc, 0)

    @pl.when(s < GROUP_EXTRA)
    def _():
        pltpu.sync_copy(rf0.at[pl.ds(0, 632 - 4 * CHUNK)],
                        acc_sh.at[pl.ds(row0 + 4 * CHUNK, 632 - 4 * CHUNK)])

    @pl.when(s >= GROUP_EXTRA)
    def _():
        pltpu.sync_copy(rf0.at[pl.ds(0, 624 - 4 * CHUNK)],
                        acc_sh.at[pl.ds(row0 + 4 * CHUNK, 624 - 4 * CHUNK)])

    plsc.subcore_barrier()

    idx_wait(0)
    pltpu.async_copy(ypk_hbm.at[sidr.at[0]], pk0, gsem0)

    def step(j, ib, rb):
        pk, rf = pks[rb], rfs[rb]
        @pl.when(j < CPT - 1)
        def _():
            idx_wait((ib + 1) % 4)
            pltpu.async_copy(ypk_hbm.at[sidr.at[(ib + 1) % 4]],
                             pks[1 - rb], gsems[1 - rb])

        pltpu.make_async_copy(ypk_hbm.at[pl.ds(0, CHUNK)], pk,
                              gsems[rb]).wait()
        @pl.when(jnp.logical_and(j >= 1, CPT * wid + j - 1 < N_CHUNKS))
        def _():
            pltpu.make_async_copy(rfs[1 - rb], acc_sh.at[didr.at[(ib + 3) % 4]],
                                  ssems[1 - rb]).wait()

        @pl.when(j < CPT - 3)
        def _():
            idx_load(j + 3, (ib + 3) % 4)

        def unpack_row(r, carry):
            for t in range(4):
                w = pk[r, pl.ds(16 * t, 16)]
                a = lax.bitcast_convert_type(w << 16, jnp.float32)
                b = lax.bitcast_convert_type(w & jnp.int32(-65536), jnp.float32)
                rf[r, pl.ds(32 * t, 16)] = a
                rf[r, pl.ds(32 * t + 16, 16)] = b
            return carry

        lax.fori_loop(0, CHUNK, unpack_row, 0)

        @pl.when(CPT * wid + j < N_CHUNKS)
        def _():
            pltpu.async_copy(rf, acc_sh.at[didr.at[ib]], ssems[rb], add=True)

    def outer(i4, carry):
        for b in range(4):
            step(i4 * 4 + b, b, b % 2)
        return carry

    lax.fori_loop(0, CPT // 4, outer, 0)

    @pl.when(CPT * wid + CPT - 1 < N_CHUNKS)
    def _():
        pltpu.make_async_copy(rf1, acc_sh.at[didr.at[3]], ssem1).wait()

    plsc.subcore_barrier()

    def wb_block(t, carry):
        r = row0 + t * CHUNK
        pltpu.sync_copy(acc_sh.at[pl.ds(r, CHUNK)], rf0)
        pltpu.sync_copy(rf0, out_hbm.at[c, pl.ds(r, CHUNK)])
        return carry

    lax.fori_loop(0, 4, wb_block, 0)
    rtail = row0 + 4 * CHUNK

    @pl.when(s < GROUP_EXTRA)
    def _():
        pltpu.sync_copy(acc_sh.at[pl.ds(rtail, 632 - 4 * CHUNK)],
                        rf0.at[pl.ds(0, 632 - 4 * CHUNK)])
        pltpu.sync_copy(rf0.at[pl.ds(0, 632 - 4 * CHUNK)],
                        out_hbm.at[c, pl.ds(rtail, 632 - 4 * CHUNK)])

    @pl.when(s >= GROUP_EXTRA)
    def _():
        pltpu.sync_copy(acc_sh.at[pl.ds(rtail, 624 - 4 * CHUNK)],
                        rf0.at[pl.ds(0, 624 - 4 * CHUNK)])
        pltpu.sync_copy(rf0.at[pl.ds(0, 624 - 4 * CHUNK)],
                        out_hbm.at[c, pl.ds(rtail, 624 - 4 * CHUNK)])


def _prep_body(degp_ref, x_ref, y0_ref, dinv_ref, dinv2_ref):
    deg = degp_ref[0, 0:1, :] + degp_ref[1, 0:1, :]
    dinv = jnp.where(deg > 0, lax.rsqrt(jnp.maximum(deg, 1e-12)), 0.0)
    dinv_c = jnp.reshape(dinv, (N_NODES, 1))
    dinv_ref[...] = dinv_c
    dinv2_ref[...] = dinv_c * dinv_c
    y0_ref[...] = x_ref[...] * dinv_c


def _scale_body(p_ref, dinv2_ref, y_ref):
    y_ref[...] = (p_ref[0] + p_ref[1]) * dinv2_ref[...]


def _final_body(p1_ref, p2_ref, p3_ref, x_ref, dinv_ref, out_ref):
    z = (p1_ref[0] + p1_ref[1] + p2_ref[0] + p2_ref[1]
         + p3_ref[0] + p3_ref[1])
    out_ref[...] = (x_ref[...] + z * dinv_ref[...]) * 0.25


def _pack(y):
    yp = y[:, _PERM].astype(jnp.bfloat16).reshape(N_NODES, D_PK, 2)
    return lax.bitcast_convert_type(yp, jnp.int32)


def kernel(x, edge_index):
    pad = (jnp.arange(N_CHUNKS_PAD * CHUNK - N_EDGES, dtype=jnp.int32)
           % N_NODES)
    srcf = jnp.concatenate([edge_index[0], pad])
    dstf = jnp.concatenate([edge_index[1], pad])
    dst2 = dstf.reshape(N_CHUNKS_PAD, CHUNK)
    zeros1 = jnp.zeros((N_NODES,), jnp.float32)

    degp = _deg_kernel(dst2, zeros1)

    f32 = jnp.float32
    nd = (N_NODES, D_FEAT)
    y0, dinv, dinv2 = pl.pallas_call(
        _prep_body,
        out_shape=(
            jax.ShapeDtypeStruct(nd, f32),
            jax.ShapeDtypeStruct((N_NODES, 1), f32),
            jax.ShapeDtypeStruct((N_NODES, 1), f32),
        ),
    )(degp, x)

    p1 = _hop_kernel(_pack(y0), srcf, dstf)
    y1 = pl.pallas_call(
        _scale_body, out_shape=jax.ShapeDtypeStruct(nd, f32),
    )(p1, dinv2)

    p2 = _hop_kernel(_pack(y1), srcf, dstf)
    y2 = pl.pallas_call(
        _scale_body, out_shape=jax.ShapeDtypeStruct(nd, f32),
    )(p2, dinv2)

    p3 = _hop_kernel(_pack(y2), srcf, dstf)
    light_out = pl.pallas_call(
        _final_body,
        out_shape=jax.ShapeDtypeStruct(nd, f32),
    )(p1, p2, p3, x, dinv)

    return (light_out[:N_USERS], light_out[N_USERS:])


# speedup vs baseline: 2.3170x; 2.3170x over previous
import functools

import jax
import jax.numpy as jnp
from jax import lax
from jax.experimental import pallas as pl
from jax.experimental.pallas import tpu as pltpu
from jax.experimental.pallas import tpu_sc as plsc

N_USERS = 4000
N_ITEMS = 6000
N_NODES = N_USERS + N_ITEMS
N_EDGES = 320000
D_FEAT = 128

NC = 2
NS = 16
NW = NC * NS
CHUNK = 128
N_CHUNKS = N_EDGES // CHUNK
CPT = 80
N_CHUNKS_PAD = CPT * NW
ROW_GROUPS = N_NODES // 8
GROUPS_PER_TILE = ROW_GROUPS // NS
GROUP_EXTRA = ROW_GROUPS - GROUPS_PER_TILE * NS

_sc_mesh = plsc.VectorSubcoreMesh(core_axis_name="c", subcore_axis_name="s")


def _row_range(s):
    row0 = 8 * (s * GROUPS_PER_TILE + jnp.minimum(s, GROUP_EXTRA))
    return pl.multiple_of(row0, 8)


@functools.partial(
    pl.kernel,
    out_type=jax.ShapeDtypeStruct((NC, 1, N_NODES), jnp.float32),
    mesh=_sc_mesh,
    scratch_types=[
        pltpu.VMEM_SHARED((N_NODES,), jnp.float32),
        pltpu.VMEM((CPT, CHUNK), jnp.int32),
        pltpu.VMEM((CHUNK,), jnp.float32),
        pltpu.SemaphoreType.DMA,
    ],
)
def _deg_kernel(dst2_hbm, zeros1_hbm, out_hbm, deg_sh, dst2d, ones_v, sem):
    c = lax.axis_index("c")
    s = lax.axis_index("s")
    wid = s * NC + c

    pltpu.sync_copy(dst2_hbm.at[pl.ds(CPT * wid, CPT)], dst2d)

    @pl.when(s == 0)
    def _():
        pltpu.sync_copy(zeros1_hbm, deg_sh)

    for j in range(CHUNK // 16):
        ones_v[pl.ds(j * 16, 16)] = jnp.ones((16,), jnp.float32)
    plsc.subcore_barrier()

    for j in range(CPT):
        @pl.when(CPT * wid + j < N_CHUNKS)
        def _():
            pltpu.async_copy(ones_v, deg_sh.at[dst2d.at[j]], sem, add=True)
    for j in range(CPT):
        @pl.when(CPT * wid + j < N_CHUNKS)
        def _():
            pltpu.make_async_copy(zeros1_hbm.at[pl.ds(0, CHUNK)], ones_v, sem).wait()

    plsc.subcore_barrier()

    @pl.when(s == 0)
    def _():
        pltpu.sync_copy(deg_sh, out_hbm.at[c, 0])


@functools.partial(
    pl.kernel,
    out_type=jax.ShapeDtypeStruct((NC, N_NODES, D_FEAT), jnp.float32),
    mesh=_sc_mesh,
    scratch_types=[
        pltpu.VMEM_SHARED((N_NODES, D_FEAT), jnp.float32),
        pltpu.VMEM((CPT, CHUNK), jnp.int32),
        pltpu.VMEM((2, CHUNK), jnp.int32),
        pltpu.VMEM((CHUNK, D_FEAT), jnp.float32),
        pltpu.VMEM((CHUNK, D_FEAT), jnp.float32),
        pltpu.SemaphoreType.DMA,
        pltpu.SemaphoreType.DMA,
        pltpu.SemaphoreType.DMA,
        pltpu.SemaphoreType.DMA,
        pltpu.SemaphoreType.DMA,
        pltpu.SemaphoreType.DMA,
    ],
)
def _hop_kernel(y_hbm, src2_hbm, dstf_hbm, out_hbm,
                acc_sh, src2d, dstr, rows0, rows1,
                sem0, sem1, dsem0, dsem1, ssem0, ssem1):
    c = lax.axis_index("c")
    s = lax.axis_index("s")
    wid = s * NC + c

    pltpu.sync_copy(src2_hbm.at[pl.ds(CPT * wid, CPT)], src2d)

    def zero_row(i, carry):
        for k in range(D_FEAT // 16):
            rows0[i, pl.ds(k * 16, 16)] = jnp.zeros((16,), jnp.float32)
        return carry

    lax.fori_loop(0, CHUNK, zero_row, 0)

    row0 = _row_range(s)

    def zero_acc(t, carry):
        pltpu.sync_copy(rows0, acc_sh.at[pl.ds(row0 + t * CHUNK, CHUNK)])
        return carry

    lax.fori_loop(0, 4, zero_acc, 0)

    @pl.when(s < GROUP_EXTRA)
    def _():
        pltpu.sync_copy(rows0.at[pl.ds(0, 632 - 4 * CHUNK)],
                        acc_sh.at[pl.ds(row0 + 4 * CHUNK, 632 - 4 * CHUNK)])

    @pl.when(s >= GROUP_EXTRA)
    def _():
        pltpu.sync_copy(rows0.at[pl.ds(0, 624 - 4 * CHUNK)],
                        acc_sh.at[pl.ds(row0 + 4 * CHUNK, 624 - 4 * CHUNK)])

    plsc.subcore_barrier()

    bufs = ((rows0, sem0, dsem0, ssem0), (rows1, sem1, dsem1, ssem1))
    pltpu.async_copy(y_hbm.at[src2d.at[0]], rows0, sem0)
    pltpu.async_copy(dstf_hbm.at[pl.ds(CPT * wid * CHUNK, CHUNK)],
                     dstr.at[0], dsem0)

    def outer(i2, carry):
        for b in range(2):
            j = i2 * 2 + b
            rows, sem, dsem, ssem = bufs[b]
            orows, osem, odsem, ossem = bufs[1 - b]
            @pl.when(j >= 1)
            def _():
                @pl.when(CPT * wid + j - 1 < N_CHUNKS)
                def _():
                    pltpu.make_async_copy(orows, acc_sh.at[dstr.at[1 - b]],
                                          ossem).wait()
            jn = jnp.minimum(j + 1, CPT - 1)
            pltpu.async_copy(y_hbm.at[src2d.at[jn]], orows, osem)
            base = pl.multiple_of((CPT * wid + jn) * CHUNK, CHUNK)
            pltpu.async_copy(dstf_hbm.at[pl.ds(base, CHUNK)],
                             dstr.at[1 - b], odsem)
            pltpu.make_async_copy(y_hbm.at[pl.ds(0, CHUNK)], rows, sem).wait()
            pltpu.make_async_copy(dstf_hbm.at[pl.ds(0, CHUNK)],
                                  dstr.at[b], dsem).wait()
            @pl.when(CPT * wid + j < N_CHUNKS)
            def _():
                pltpu.async_copy(rows, acc_sh.at[dstr.at[b]], ssem, add=True)
        return carry

    lax.fori_loop(0, CPT // 2, outer, 0)
    @pl.when(CPT * wid + CPT - 1 < N_CHUNKS)
    def _():
        pltpu.make_async_copy(rows1, acc_sh.at[dstr.at[1]], ssem1).wait()

    pltpu.make_async_copy(y_hbm.at[pl.ds(0, CHUNK)], rows0, sem0).wait()
    pltpu.make_async_copy(dstf_hbm.at[pl.ds(0, CHUNK)], dstr.at[0],
                          dsem0).wait()

    plsc.subcore_barrier()

    def wb_block(t, carry):
        r = row0 + t * CHUNK
        pltpu.sync_copy(acc_sh.at[pl.ds(r, CHUNK)], rows0)
        pltpu.sync_copy(rows0, out_hbm.at[c, pl.ds(r, CHUNK)])
        return carry

    lax.fori_loop(0, 4, wb_block, 0)
    rtail = row0 + 4 * CHUNK

    @pl.when(s < GROUP_EXTRA)
    def _():
        pltpu.sync_copy(acc_sh.at[pl.ds(rtail, 632 - 4 * CHUNK)],
                        rows0.at[pl.ds(0, 632 - 4 * CHUNK)])
        pltpu.sync_copy(rows0.at[pl.ds(0, 632 - 4 * CHUNK)],
                        out_hbm.at[c, pl.ds(rtail, 632 - 4 * CHUNK)])

    @pl.when(s >= GROUP_EXTRA)
    def _():
        pltpu.sync_copy(acc_sh.at[pl.ds(rtail, 624 - 4 * CHUNK)],
                        rows0.at[pl.ds(0, 624 - 4 * CHUNK)])
        pltpu.sync_copy(rows0.at[pl.ds(0, 624 - 4 * CHUNK)],
                        out_hbm.at[c, pl.ds(rtail, 624 - 4 * CHUNK)])


def _prep_body(degp_ref, x_ref, y0_ref, dinv_ref, dinv2_ref):
    deg = degp_ref[0, 0:1, :] + degp_ref[1, 0:1, :]
    dinv = jnp.where(deg > 0, lax.rsqrt(jnp.maximum(deg, 1e-12)), 0.0)
    dinv_c = jnp.reshape(dinv, (N_NODES, 1))
    dinv_ref[...] = dinv_c
    dinv2_ref[...] = dinv_c * dinv_c
    y0_ref[...] = x_ref[...] * dinv_c


def _scale_body(p_ref, dinv2_ref, y_ref):
    y_ref[...] = (p_ref[0] + p_ref[1]) * dinv2_ref[...]


def _final_a_body(p1_ref, p2_ref, x_ref, dinv_ref, m_ref):
    z12 = p1_ref[0] + p1_ref[1] + p2_ref[0] + p2_ref[1]
    m_ref[...] = x_ref[...] + z12 * dinv_ref[...]


def _final_b_body(p3_ref, m_ref, dinv_ref, out_ref):
    z3 = p3_ref[0] + p3_ref[1]
    out_ref[...] = (m_ref[...] + z3 * dinv_ref[...]) * 0.25


def kernel(x, edge_index):
    pad = (jnp.arange(N_CHUNKS_PAD * CHUNK - N_EDGES, dtype=jnp.int32)
           % N_NODES)
    src2 = jnp.concatenate([edge_index[0], pad]).reshape(N_CHUNKS_PAD, CHUNK)
    dstf = jnp.concatenate([edge_index[1], pad])
    dst2 = dstf.reshape(N_CHUNKS_PAD, CHUNK)
    zeros1 = jnp.zeros((N_NODES,), jnp.float32)

    degp = _deg_kernel(dst2, zeros1)

    f32 = jnp.float32
    nd = (N_NODES, D_FEAT)
    y0, dinv, dinv2 = pl.pallas_call(
        _prep_body,
        out_shape=(
            jax.ShapeDtypeStruct(nd, f32),
            jax.ShapeDtypeStruct((N_NODES, 1), f32),
            jax.ShapeDtypeStruct((N_NODES, 1), f32),
        ),
    )(degp, x)

    p1 = _hop_kernel(y0, src2, dstf)
    y1 = pl.pallas_call(
        _scale_body, out_shape=jax.ShapeDtypeStruct(nd, f32),
    )(p1, dinv2)

    p2 = _hop_kernel(y1, src2, dstf)
    y2 = pl.pallas_call(
        _scale_body, out_shape=jax.ShapeDtypeStruct(nd, f32),
    )(p2, dinv2)

    p3 = _hop_kernel(y2, src2, dstf)
    m12 = pl.pallas_call(
        _final_a_body, out_shape=jax.ShapeDtypeStruct(nd, f32),
    )(p1, p2, x, dinv)
    light_out = pl.pallas_call(
        _final_b_body,
        out_shape=jax.ShapeDtypeStruct(nd, f32),
    )(p3, m12, dinv)

    return (light_out[:N_USERS], light_out[N_USERS:])
